# baseline (speedup 1.0000x reference)
import jax
import jax.numpy as jnp
from jax import lax
from jax.experimental import pallas as pl
from jax.experimental.pallas import tpu as pltpu
from jax.experimental.pallas import tpu_sc as plsc

N = 10000
E = 160000
D_IN = 128
D_H = 512
D_OUT = 128
EPS = 0.1

NT = 10112
NTD = 10240
NTILE = 16
RPTD = NTD // NTILE
BS = 128
BPT = 80
NB = NTILE * BPT
EP = NB * BS
NBR = E // BS
NCHUNK = 4
DC = D_H // NCHUNK

THR = 8832
A1 = 8960
Z1 = A1 // NTILE
W1R = THR // NTILE
P2R = NT - THR
A2 = P2R + 128
Z2 = A2 // NTILE
W2R = P2R // NTILE
ZB = 32

R = 2000
GRID = N // R

_sc_mesh = plsc.VectorSubcoreMesh(core_axis_name="c", subcore_axis_name="s")


def _prop_body(t_hbm, srcp, dstp, slp, cnp, out_hbm, src_idx, raw_idx,
               dbuf, rows, zbuf, slv, cnv, acc, sem, ssem):
    c = lax.axis_index("c")
    s = lax.axis_index("s")
    zv = jnp.zeros((16,), jnp.float32)

    def _zrow(i, carry):
        for j in range(DC // 16):
            zbuf[i, pl.ds(j * 16, 16)] = zv
        return carry

    lax.fori_loop(0, ZB, _zrow, 0)

    pltpu.sync_copy(srcp.at[s], src_idx)
    pltpu.sync_copy(dstp.at[s], raw_idx)
    pltpu.sync_copy(slp.at[s], slv)
    pltpu.sync_copy(cnp.at[s], cnv)

    def _chunk(jc, carry):
        ch = c * (NCHUNK // 2) + jc
        for p in range(2):
            zrows = Z1 if p == 0 else Z2
            for off in range(0, zrows, ZB):
                nz = min(ZB, zrows - off)
                pltpu.sync_copy(zbuf.at[pl.ds(0, nz)],
                                acc.at[pl.ds(s * zrows + off, nz)])
            plsc.subcore_barrier()

            cnt = cnv[p, pl.ds(0, 16)][0]
            b0 = slv[p, pl.ds(0, 16)][0]
            pltpu.async_copy(t_hbm.at[ch].at[src_idx.at[b0]], rows.at[0],
                             sem.at[0])

            def _batch(i, b):
                par = i % 2
                npar = 1 - par

                @pl.when(i > 0)
                def _():
                    pltpu.make_async_copy(rows.at[npar],
                                          acc.at[pl.ds(0, BS)],
                                          ssem.at[npar]).wait()

                nb = slv[p, pl.ds(i + 1, 16)][0]
                pltpu.async_copy(t_hbm.at[ch].at[src_idx.at[nb]],
                                 rows.at[npar], sem.at[npar])
                for jj in range(BS // 16):
                    v = raw_idx[b, pl.ds(jj * 16, 16)]
                    tr = (lax.iota(jnp.int32, 16) + 4 * jj) % 64
                    if p == 0:
                        dbuf[par, pl.ds(jj * 16, 16)] = jnp.where(
                            v < THR, v, THR + tr)
                    else:
                        dbuf[par, pl.ds(jj * 16, 16)] = jnp.where(
                            v >= THR, v - THR, P2R + tr)
                pltpu.make_async_copy(t_hbm.at[ch].at[src_idx.at[b]],
                                      rows.at[par], sem.at[par]).wait()
                pltpu.async_copy(rows.at[par], acc.at[dbuf.at[par]],
                                 ssem.at[par], add=True)
                return nb

            bl = lax.fori_loop(0, cnt, _batch, b0)
            parl = cnt % 2

            @pl.when(cnt > 0)
            def _():
                pltpu.make_async_copy(rows.at[1 - parl],
                                      acc.at[pl.ds(0, BS)],
                                      ssem.at[1 - parl]).wait()

            pltpu.make_async_copy(t_hbm.at[ch].at[src_idx.at[bl]],
                                  rows.at[parl], sem.at[parl]).wait()
            plsc.subcore_barrier()
            if p == 0:
                pltpu.sync_copy(acc.at[pl.ds(s * W1R, W1R)],
                                out_hbm.at[ch].at[pl.ds(s * W1R, W1R)])
            else:
                pltpu.sync_copy(acc.at[pl.ds(s * W2R, W2R)],
                                out_hbm.at[ch].at[pl.ds(THR + s * W2R, W2R)])
            plsc.subcore_barrier()
        return carry

    lax.fori_loop(0, NCHUNK // 2, _chunk, 0)


def _propagate(table, srcp, dstp, slp, cnp):
    return pl.kernel(
        _prop_body,
        out_type=jax.ShapeDtypeStruct((NCHUNK, NT, DC), jnp.float32),
        mesh=_sc_mesh,
        scratch_types=[
            pltpu.VMEM((BPT, BS), jnp.int32),
            pltpu.VMEM((BPT, BS), jnp.int32),
            pltpu.VMEM((2, BS), jnp.int32),
            pltpu.VMEM((2, BS, DC), jnp.float32),
            pltpu.VMEM((ZB, DC), jnp.float32),
            pltpu.VMEM((2, BS), jnp.int32),
            pltpu.VMEM((2, 16), jnp.int32),
            pltpu.VMEM_SHARED((A1, DC), jnp.float32),
            pltpu.SemaphoreType.DMA((2,)),
            pltpu.SemaphoreType.DMA((2,)),
        ],
    )(table, srcp, dstp, slp, cnp)


def _deg_body(dstp, deg_out, dst_idx, ones_v, zbuf, acc):
    c = lax.axis_index("c")
    s = lax.axis_index("s")
    zv = jnp.zeros((16,), jnp.float32)
    ov = jnp.ones((16,), jnp.float32)
    for i in range(BS // 16):
        ones_v[pl.ds(i * 16, 16)] = ov

    def _z(i, carry):
        zbuf[pl.ds(i * 16, 16)] = zv
        return carry

    lax.fori_loop(0, RPTD // 16, _z, 0)

    pltpu.sync_copy(dstp.at[s], dst_idx)
    pltpu.sync_copy(zbuf, acc.at[pl.ds(s * RPTD, RPTD)])
    plsc.subcore_barrier()

    def _b(b, carry):
        pltpu.sync_copy(ones_v, acc.at[dst_idx.at[c * (BPT // 2) + b]],
                        add=True)
        return carry

    lax.fori_loop(0, BPT // 2, _b, 0)
    plsc.subcore_barrier()
    pltpu.sync_copy(acc.at[pl.ds(s * RPTD, RPTD)],
                    deg_out.at[c].at[pl.ds(s * RPTD, RPTD)])


def _degrees(dstp):
    return pl.kernel(
        _deg_body,
        out_type=jax.ShapeDtypeStruct((2, NTD), jnp.float32),
        mesh=_sc_mesh,
        scratch_types=[
            pltpu.VMEM((BPT, BS), jnp.int32),
            pltpu.VMEM((BS,), jnp.float32),
            pltpu.VMEM((RPTD,), jnp.float32),
            pltpu.VMEM_SHARED((NTD,), jnp.float32),
        ],
    )(dstp)


def _dis_body(degf_ref, dis_ref):
    d = degf_ref[pl.ds(0, NTD)] + degf_ref[pl.ds(NTD, NTD)]
    dis_ref[...] = jnp.where(d > 0.0, lax.rsqrt(jnp.maximum(d, 1e-12)), 0.0)


_dis_call = pl.pallas_call(
    _dis_body,
    out_shape=jax.ShapeDtypeStruct((NTD,), jnp.float32),
)


def _emb_body(x_ref, w_ref, b_ref, h_ref):
    h_ref[...] = (jnp.dot(x_ref[...], w_ref[...],
                          preferred_element_type=jnp.float32) + b_ref[...])


_emb_call = pl.pallas_call(
    _emb_body,
    grid=(GRID,),
    in_specs=[
        pl.BlockSpec((R, D_IN), lambda i: (i, 0)),
        pl.BlockSpec((D_IN, D_H), lambda i: (0, 0)),
        pl.BlockSpec((D_H,), lambda i: (0,)),
    ],
    out_specs=pl.BlockSpec((R, D_H), lambda i: (i, 0)),
    out_shape=jax.ShapeDtypeStruct((N, D_H), jnp.float32),
)


def _scale_body(h_ref, dis_ref, u_ref):
    hs = h_ref[...] * dis_ref[...]
    for cc in range(NCHUNK):
        u_ref[cc] = hs[:, cc * DC:(cc + 1) * DC]


_scale_call = pl.pallas_call(
    _scale_body,
    grid=(GRID,),
    in_specs=[
        pl.BlockSpec((R, D_H), lambda i: (i, 0)),
        pl.BlockSpec((R, 1), lambda i: (i, 0)),
    ],
    out_specs=pl.BlockSpec((NCHUNK, R, DC), lambda i: (0, i, 0)),
    out_shape=jax.ShapeDtypeStruct((NCHUNK, NT, DC), jnp.float32),
)


def _scale2_body(p_ref, dis_ref, v_ref):
    dis = dis_ref[...]
    for cc in range(NCHUNK):
        v_ref[cc] = p_ref[cc] * (dis * dis)


_scale2_call = pl.pallas_call(
    _scale2_body,
    grid=(GRID,),
    in_specs=[
        pl.BlockSpec((NCHUNK, R, DC), lambda i: (0, i, 0)),
        pl.BlockSpec((R, 1), lambda i: (i, 0)),
    ],
    out_specs=pl.BlockSpec((NCHUNK, R, DC), lambda i: (0, i, 0)),
    out_shape=jax.ShapeDtypeStruct((NCHUNK, NT, DC), jnp.float32),
)


def _mm0_body(h_ref, w_ref, s_ref):
    s_ref[...] = jnp.dot(h_ref[...], w_ref[...],
                         preferred_element_type=jnp.float32)


_mm0_call = pl.pallas_call(
    _mm0_body,
    grid=(GRID,),
    in_specs=[
        pl.BlockSpec((R, D_H), lambda i: (i, 0)),
        pl.BlockSpec((D_H, D_H), lambda i: (0, 0)),
    ],
    out_specs=pl.BlockSpec((R, D_H), lambda i: (i, 0)),
    out_shape=jax.ShapeDtypeStruct((N, D_H), jnp.float32),
)


def _mid_body(p_ref, dis_ref, sin_ref, w_ref, sout_ref):
    p = jnp.concatenate([p_ref[cc] for cc in range(NCHUNK)], axis=1)
    t = p * dis_ref[...]
    sout_ref[...] = sin_ref[...] + jnp.dot(t, w_ref[...],
                                           preferred_element_type=jnp.float32)


_mid_call = pl.pallas_call(
    _mid_body,
    grid=(GRID,),
    in_specs=[
        pl.BlockSpec((NCHUNK, R, DC), lambda i: (0, i, 0)),
        pl.BlockSpec((R, 1), lambda i: (i, 0)),
        pl.BlockSpec((R, D_H), lambda i: (i, 0)),
        pl.BlockSpec((D_H, D_H), lambda i: (0, 0)),
    ],
    out_specs=pl.BlockSpec((R, D_H), lambda i: (i, 0)),
    out_shape=jax.ShapeDtypeStruct((N, D_H), jnp.float32),
)


def _post_body(p_ref, dis_ref, sin_ref, w_ref, b_ref, h_ref, hout_ref):
    p = jnp.concatenate([p_ref[cc] for cc in range(NCHUNK)], axis=1)
    t = p * dis_ref[...]
    conv = (sin_ref[...]
            + jnp.dot(t, w_ref[...], preferred_element_type=jnp.float32)
            + b_ref[...])
    hout_ref[...] = h_ref[...] + EPS * jnp.tanh(conv)


_post_call = pl.pallas_call(
    _post_body,
    grid=(GRID,),
    in_specs=[
        pl.BlockSpec((NCHUNK, R, DC), lambda i: (0, i, 0)),
        pl.BlockSpec((R, 1), lambda i: (i, 0)),
        pl.BlockSpec((R, D_H), lambda i: (i, 0)),
        pl.BlockSpec((D_H, D_H), lambda i: (0, 0)),
        pl.BlockSpec((D_H,), lambda i: (0,)),
        pl.BlockSpec((R, D_H), lambda i: (i, 0)),
    ],
    out_specs=pl.BlockSpec((R, D_H), lambda i: (i, 0)),
    out_shape=jax.ShapeDtypeStruct((N, D_H), jnp.float32),
)


def _ro_body(h_ref, w_ref, b_ref, y_ref):
    y_ref[...] = (jnp.dot(h_ref[...], w_ref[...],
                          preferred_element_type=jnp.float32) + b_ref[...])


_ro_call = pl.pallas_call(
    _ro_body,
    grid=(GRID,),
    in_specs=[
        pl.BlockSpec((R, D_H), lambda i: (i, 0)),
        pl.BlockSpec((D_H, D_OUT), lambda i: (0, 0)),
        pl.BlockSpec((D_OUT,), lambda i: (0,)),
    ],
    out_specs=pl.BlockSpec((R, D_OUT), lambda i: (i, 0)),
    out_shape=jax.ShapeDtypeStruct((N, D_OUT), jnp.float32),
)


def kernel(x, edge_index, delta_t, emb_w, emb_b, lins_w, conv_b, ro_w, ro_b):
    src = edge_index[0]
    dst = edge_index[1]
    bit = (dst >= THR).astype(jnp.int32)
    below = jnp.cumsum(1 - bit)
    above = jnp.cumsum(bit)
    pos = jnp.where(bit == 1, below[-1] + above - 1, below - 1)
    fill = (N + (jnp.arange(EP - E, dtype=jnp.int32) % 64)).astype(jnp.int32)
    src_part = jnp.concatenate(
        [jnp.zeros((E,), jnp.int32).at[pos].set(src), fill])
    dst_part = jnp.concatenate(
        [jnp.zeros((E,), jnp.int32).at[pos].set(dst), fill])
    bmin = dst_part.reshape(NB, BS).min(axis=1)
    bmax = dst_part.reshape(NB, BS).max(axis=1)
    real = jnp.arange(NB) < NBR
    f1 = ((bmin < THR) & real).astype(jnp.int32)
    f2 = ((bmax >= THR) & real).astype(jnp.int32)
    af = jnp.stack([f1, f2], 1).reshape(BPT, NTILE, 2).transpose(1, 2, 0)
    slp = jnp.pad(jnp.argsort(1 - af, axis=-1),
                  ((0, 0), (0, 0), (0, BS - BPT))).astype(jnp.int32)
    cnp = jnp.pad(af.sum(-1, keepdims=True),
                  ((0, 0), (0, 0), (0, 15))).astype(jnp.int32)

    def _tiled(a):
        return a.reshape(BPT, NTILE, BS).transpose(1, 0, 2)

    srcp = _tiled(src_part)
    dstp = _tiled(dst_part)

    degp = _degrees(dstp)
    dis = _dis_call(degp.reshape(2 * NTD))
    dis2 = dis.reshape(NTD, 1)

    h0 = _emb_call(x, emb_w, emb_b)

    w0 = lins_w[0]
    w1 = lins_w[1]
    w2 = lins_w[2]

    def _step(_, h):
        u = _scale_call(h, dis2)
        p1 = _propagate(u, srcp, dstp, slp, cnp)
        s0 = _mm0_call(h, w0)
        v = _scale2_call(p1, dis2)
        p2 = _propagate(v, srcp, dstp, slp, cnp)
        s1 = _mid_call(p1, dis2, s0, w1)
        return _post_call(p2, dis2, s1, w2, conv_b, h)

    h = lax.fori_loop(0, delta_t, _step, h0)
    y = _ro_call(h, ro_w, ro_b)
    return (y, h)


# speedup vs baseline: 1.0039x; 1.0039x over previous
import jax
import jax.numpy as jnp
from jax import lax
from jax.experimental import pallas as pl
from jax.experimental.pallas import tpu as pltpu
from jax.experimental.pallas import tpu_sc as plsc

N = 10000
E = 160000
D_IN = 128
D_H = 512
D_OUT = 128
EPS = 0.1

NT = 10112
NTD = 10240
NTILE = 16
RPTD = NTD // NTILE
BS = 128
BPT = 80
NB = NTILE * BPT
EP = NB * BS
NBR = E // BS
NCHUNK = 4
DC = D_H // NCHUNK

THR = 8832
A1 = 8960
Z1 = A1 // NTILE
W1R = THR // NTILE
P2R = NT - THR
A2 = P2R + 128
Z2 = A2 // NTILE
W2R = P2R // NTILE
ZB = 32

R = 2000
GRID = N // R

_sc_mesh = plsc.VectorSubcoreMesh(core_axis_name="c", subcore_axis_name="s")


def _prop_body(t_hbm, srcp, dstp, slp, cnp, out_hbm, src_idx, raw_idx,
               dbuf, rows, zbuf, slv, cnv, acc, sem, ssem):
    c = lax.axis_index("c")
    s = lax.axis_index("s")
    zv = jnp.zeros((16,), jnp.float32)

    def _zrow(i, carry):
        for j in range(DC // 16):
            zbuf[i, pl.ds(j * 16, 16)] = zv
        return carry

    lax.fori_loop(0, ZB, _zrow, 0)

    pltpu.sync_copy(srcp.at[s], src_idx)
    pltpu.sync_copy(dstp.at[s], raw_idx)
    pltpu.sync_copy(slp.at[s], slv)
    pltpu.sync_copy(cnp.at[s], cnv)

    def _chunk(jc, carry):
        ch = c * (NCHUNK // 2) + jc
        for p in range(2):
            zrows = Z1 if p == 0 else Z2
            for off in range(0, zrows, ZB):
                nz = min(ZB, zrows - off)
                pltpu.sync_copy(zbuf.at[pl.ds(0, nz)],
                                acc.at[pl.ds(s * zrows + off, nz)])
            plsc.subcore_barrier()

            cnt = cnv[p, pl.ds(0, 16)][0]
            b0 = slv[p, pl.ds(0, 16)][0]
            pltpu.async_copy(t_hbm.at[ch].at[src_idx.at[b0]], rows.at[0],
                             sem.at[0])

            def _batch(i, b):
                par = i % 2
                npar = 1 - par

                @pl.when(i > 0)
                def _():
                    pltpu.make_async_copy(rows.at[npar],
                                          acc.at[pl.ds(0, BS)],
                                          ssem.at[npar]).wait()

                nb = slv[p, pl.ds(i + 1, 16)][0]
                pltpu.async_copy(t_hbm.at[ch].at[src_idx.at[nb]],
                                 rows.at[npar], sem.at[npar])
                for jj in range(BS // 16):
                    v = raw_idx[b, pl.ds(jj * 16, 16)]
                    tr = (lax.iota(jnp.int32, 16) + 4 * jj) % 64
                    if p == 0:
                        dbuf[par, pl.ds(jj * 16, 16)] = jnp.where(
                            v < THR, v, THR + tr)
                    else:
                        dbuf[par, pl.ds(jj * 16, 16)] = jnp.where(
                            v >= THR, v - THR, P2R + tr)
                pltpu.make_async_copy(t_hbm.at[ch].at[src_idx.at[b]],
                                      rows.at[par], sem.at[par]).wait()
                pltpu.async_copy(rows.at[par], acc.at[dbuf.at[par]],
                                 ssem.at[par], add=True)
                return nb

            bl = lax.fori_loop(0, cnt, _batch, b0)
            parl = cnt % 2

            @pl.when(cnt > 0)
            def _():
                pltpu.make_async_copy(rows.at[1 - parl],
                                      acc.at[pl.ds(0, BS)],
                                      ssem.at[1 - parl]).wait()

            pltpu.make_async_copy(t_hbm.at[ch].at[src_idx.at[bl]],
                                  rows.at[parl], sem.at[parl]).wait()
            plsc.subcore_barrier()
            if p == 0:
                pltpu.sync_copy(acc.at[pl.ds(s * W1R, W1R)],
                                out_hbm.at[ch].at[pl.ds(s * W1R, W1R)])
            else:
                pltpu.sync_copy(acc.at[pl.ds(s * W2R, W2R)],
                                out_hbm.at[ch].at[pl.ds(THR + s * W2R, W2R)])
            plsc.subcore_barrier()
        return carry

    lax.fori_loop(0, NCHUNK // 2, _chunk, 0)


def _propagate(table, srcp, dstp, slp, cnp):
    return pl.kernel(
        _prop_body,
        out_type=jax.ShapeDtypeStruct((NCHUNK, NT, DC), jnp.float32),
        mesh=_sc_mesh,
        scratch_types=[
            pltpu.VMEM((BPT, BS), jnp.int32),
            pltpu.VMEM((BPT, BS), jnp.int32),
            pltpu.VMEM((2, BS), jnp.int32),
            pltpu.VMEM((2, BS, DC), jnp.float32),
            pltpu.VMEM((ZB, DC), jnp.float32),
            pltpu.VMEM((2, BS), jnp.int32),
            pltpu.VMEM((2, 16), jnp.int32),
            pltpu.VMEM_SHARED((A1, DC), jnp.float32),
            pltpu.SemaphoreType.DMA((2,)),
            pltpu.SemaphoreType.DMA((2,)),
        ],
    )(table, srcp, dstp, slp, cnp)


def _deg_body(dstp, deg_out, dst_idx, ones_v, zbuf, acc):
    c = lax.axis_index("c")
    s = lax.axis_index("s")
    zv = jnp.zeros((16,), jnp.float32)
    ov = jnp.ones((16,), jnp.float32)
    for i in range(BS // 16):
        ones_v[pl.ds(i * 16, 16)] = ov

    def _z(i, carry):
        zbuf[pl.ds(i * 16, 16)] = zv
        return carry

    lax.fori_loop(0, RPTD // 16, _z, 0)

    pltpu.sync_copy(dstp.at[s], dst_idx)
    pltpu.sync_copy(zbuf, acc.at[pl.ds(s * RPTD, RPTD)])
    plsc.subcore_barrier()

    def _b(b, carry):
        pltpu.sync_copy(ones_v, acc.at[dst_idx.at[c * (BPT // 2) + b]],
                        add=True)
        return carry

    lax.fori_loop(0, BPT // 2, _b, 0)
    plsc.subcore_barrier()
    pltpu.sync_copy(acc.at[pl.ds(s * RPTD, RPTD)],
                    deg_out.at[c].at[pl.ds(s * RPTD, RPTD)])


def _degrees(dstp):
    return pl.kernel(
        _deg_body,
        out_type=jax.ShapeDtypeStruct((2, NTD), jnp.float32),
        mesh=_sc_mesh,
        scratch_types=[
            pltpu.VMEM((BPT, BS), jnp.int32),
            pltpu.VMEM((BS,), jnp.float32),
            pltpu.VMEM((RPTD,), jnp.float32),
            pltpu.VMEM_SHARED((NTD,), jnp.float32),
        ],
    )(dstp)


def _dis_body(degf_ref, dis_ref):
    d = degf_ref[pl.ds(0, NTD)] + degf_ref[pl.ds(NTD, NTD)]
    dis_ref[...] = jnp.where(d > 0.0, lax.rsqrt(jnp.maximum(d, 1e-12)), 0.0)


_dis_call = pl.pallas_call(
    _dis_body,
    out_shape=jax.ShapeDtypeStruct((NTD,), jnp.float32),
)


def _emb_body(x_ref, w_ref, b_ref, h_ref):
    h_ref[...] = (jnp.dot(x_ref[...], w_ref[...],
                          preferred_element_type=jnp.float32) + b_ref[...])


_emb_call = pl.pallas_call(
    _emb_body,
    grid=(GRID,),
    in_specs=[
        pl.BlockSpec((R, D_IN), lambda i: (i, 0)),
        pl.BlockSpec((D_IN, D_H), lambda i: (0, 0)),
        pl.BlockSpec((D_H,), lambda i: (0,)),
    ],
    out_specs=pl.BlockSpec((R, D_H), lambda i: (i, 0)),
    out_shape=jax.ShapeDtypeStruct((N, D_H), jnp.float32),
)


def _scale_body(h_ref, dis_ref, u_ref):
    hs = h_ref[...] * dis_ref[...]
    for cc in range(NCHUNK):
        u_ref[cc] = hs[:, cc * DC:(cc + 1) * DC]


_scale_call = pl.pallas_call(
    _scale_body,
    grid=(GRID,),
    in_specs=[
        pl.BlockSpec((R, D_H), lambda i: (i, 0)),
        pl.BlockSpec((R, 1), lambda i: (i, 0)),
    ],
    out_specs=pl.BlockSpec((NCHUNK, R, DC), lambda i: (0, i, 0)),
    out_shape=jax.ShapeDtypeStruct((NCHUNK, NT, DC), jnp.float32),
)


def _mm0_body(h_ref, w_ref, s_ref):
    s_ref[...] = jnp.dot(h_ref[...], w_ref[...],
                         preferred_element_type=jnp.float32)


_mm0_call = pl.pallas_call(
    _mm0_body,
    grid=(GRID,),
    in_specs=[
        pl.BlockSpec((R, D_H), lambda i: (i, 0)),
        pl.BlockSpec((D_H, D_H), lambda i: (0, 0)),
    ],
    out_specs=pl.BlockSpec((R, D_H), lambda i: (i, 0)),
    out_shape=jax.ShapeDtypeStruct((N, D_H), jnp.float32),
)


def _mid_body(p_ref, dis_ref, sin_ref, w_ref, sout_ref, v_ref):
    dis = dis_ref[...]
    p = jnp.concatenate([p_ref[cc] for cc in range(NCHUNK)], axis=1)
    t = p * dis
    v = t * dis
    for cc in range(NCHUNK):
        v_ref[cc] = v[:, cc * DC:(cc + 1) * DC]
    sout_ref[...] = sin_ref[...] + jnp.dot(t, w_ref[...],
                                           preferred_element_type=jnp.float32)


_mid_call = pl.pallas_call(
    _mid_body,
    grid=(GRID,),
    in_specs=[
        pl.BlockSpec((NCHUNK, R, DC), lambda i: (0, i, 0)),
        pl.BlockSpec((R, 1), lambda i: (i, 0)),
        pl.BlockSpec((R, D_H), lambda i: (i, 0)),
        pl.BlockSpec((D_H, D_H), lambda i: (0, 0)),
    ],
    out_specs=[
        pl.BlockSpec((R, D_H), lambda i: (i, 0)),
        pl.BlockSpec((NCHUNK, R, DC), lambda i: (0, i, 0)),
    ],
    out_shape=[
        jax.ShapeDtypeStruct((N, D_H), jnp.float32),
        jax.ShapeDtypeStruct((NCHUNK, NT, DC), jnp.float32),
    ],
)


def _post_body(p_ref, dis_ref, sin_ref, w_ref, b_ref, h_ref, w0_ref,
               hout_ref, u_ref, s0_ref):
    dis = dis_ref[...]
    p = jnp.concatenate([p_ref[cc] for cc in range(NCHUNK)], axis=1)
    t = p * dis
    conv = (sin_ref[...]
            + jnp.dot(t, w_ref[...], preferred_element_type=jnp.float32)
            + b_ref[...])
    hn = h_ref[...] + EPS * jnp.tanh(conv)
    hout_ref[...] = hn
    hs = hn * dis
    for cc in range(NCHUNK):
        u_ref[cc] = hs[:, cc * DC:(cc + 1) * DC]
    s0_ref[...] = jnp.dot(hn, w0_ref[...], preferred_element_type=jnp.float32)


_post_call = pl.pallas_call(
    _post_body,
    grid=(GRID,),
    in_specs=[
        pl.BlockSpec((NCHUNK, R, DC), lambda i: (0, i, 0)),
        pl.BlockSpec((R, 1), lambda i: (i, 0)),
        pl.BlockSpec((R, D_H), lambda i: (i, 0)),
        pl.BlockSpec((D_H, D_H), lambda i: (0, 0)),
        pl.BlockSpec((D_H,), lambda i: (0,)),
        pl.BlockSpec((R, D_H), lambda i: (i, 0)),
        pl.BlockSpec((D_H, D_H), lambda i: (0, 0)),
    ],
    out_specs=[
        pl.BlockSpec((R, D_H), lambda i: (i, 0)),
        pl.BlockSpec((NCHUNK, R, DC), lambda i: (0, i, 0)),
        pl.BlockSpec((R, D_H), lambda i: (i, 0)),
    ],
    out_shape=[
        jax.ShapeDtypeStruct((N, D_H), jnp.float32),
        jax.ShapeDtypeStruct((NCHUNK, NT, DC), jnp.float32),
        jax.ShapeDtypeStruct((N, D_H), jnp.float32),
    ],
)


def _ro_body(h_ref, w_ref, b_ref, y_ref):
    y_ref[...] = (jnp.dot(h_ref[...], w_ref[...],
                          preferred_element_type=jnp.float32) + b_ref[...])


_ro_call = pl.pallas_call(
    _ro_body,
    grid=(GRID,),
    in_specs=[
        pl.BlockSpec((R, D_H), lambda i: (i, 0)),
        pl.BlockSpec((D_H, D_OUT), lambda i: (0, 0)),
        pl.BlockSpec((D_OUT,), lambda i: (0,)),
    ],
    out_specs=pl.BlockSpec((R, D_OUT), lambda i: (i, 0)),
    out_shape=jax.ShapeDtypeStruct((N, D_OUT), jnp.float32),
)


def kernel(x, edge_index, delta_t, emb_w, emb_b, lins_w, conv_b, ro_w, ro_b):
    src = edge_index[0]
    dst = edge_index[1]
    bit = (dst >= THR).astype(jnp.int32)
    below = jnp.cumsum(1 - bit)
    above = jnp.cumsum(bit)
    pos = jnp.where(bit == 1, below[-1] + above - 1, below - 1)
    fill = (N + (jnp.arange(EP - E, dtype=jnp.int32) % 64)).astype(jnp.int32)
    src_part = jnp.concatenate(
        [jnp.zeros((E,), jnp.int32).at[pos].set(src), fill])
    dst_part = jnp.concatenate(
        [jnp.zeros((E,), jnp.int32).at[pos].set(dst), fill])
    bmin = dst_part.reshape(NB, BS).min(axis=1)
    bmax = dst_part.reshape(NB, BS).max(axis=1)
    real = jnp.arange(NB) < NBR
    f1 = ((bmin < THR) & real).astype(jnp.int32)
    f2 = ((bmax >= THR) & real).astype(jnp.int32)
    af = jnp.stack([f1, f2], 1).reshape(BPT, NTILE, 2).transpose(1, 2, 0)
    slp = jnp.pad(jnp.argsort(1 - af, axis=-1),
                  ((0, 0), (0, 0), (0, BS - BPT))).astype(jnp.int32)
    cnp = jnp.pad(af.sum(-1, keepdims=True),
                  ((0, 0), (0, 0), (0, 15))).astype(jnp.int32)

    def _tiled(a):
        return a.reshape(BPT, NTILE, BS).transpose(1, 0, 2)

    srcp = _tiled(src_part)
    dstp = _tiled(dst_part)

    degp = _degrees(dstp)
    dis = _dis_call(degp.reshape(2 * NTD))
    dis2 = dis.reshape(NTD, 1)

    h0 = _emb_call(x, emb_w, emb_b)

    w0 = lins_w[0]
    w1 = lins_w[1]
    w2 = lins_w[2]

    u0 = _scale_call(h0, dis2)
    s00 = _mm0_call(h0, w0)

    def _step(_, carry):
        h, u, s0 = carry
        p1 = _propagate(u, srcp, dstp, slp, cnp)
        s1, v = _mid_call(p1, dis2, s0, w1)
        p2 = _propagate(v, srcp, dstp, slp, cnp)
        return _post_call(p2, dis2, s1, w2, conv_b, h, w0)

    h, _, _ = lax.fori_loop(0, delta_t, _step, (h0, u0, s00))
    y = _ro_call(h, ro_w, ro_b)
    return (y, h)


# speedup vs baseline: 1.4102x; 1.4048x over previous
import jax
import jax.numpy as jnp
from jax import lax
from jax.experimental import pallas as pl
from jax.experimental.pallas import tpu as pltpu
from jax.experimental.pallas import tpu_sc as plsc

N = 10000
E = 160000
D_IN = 128
D_H = 512
D_OUT = 128
EPS = 0.1

NT = 10112
NTD = 10240
NTILE = 16
RPTD = NTD // NTILE
BS = 128
BPT = 80
NB = NTILE * BPT
EP = NB * BS
NBR = E // BS
NCHUNK = 4
DC = D_H // NCHUNK

THR = 8832
A1 = 8960
Z1 = A1 // NTILE
W1R = THR // NTILE
P2R = NT - THR
A2 = P2R + 128
Z2 = A2 // NTILE
W2R = P2R // NTILE
ZB = 32

R = 2000
GRID = N // R

_sc_mesh = plsc.VectorSubcoreMesh(core_axis_name="c", subcore_axis_name="s")


def _prop_body(t_hbm, srcp, dstp, slp, cnp, out_hbm, src_idx, raw_idx,
               dbuf, rows, zbuf, slv, cnv, acc, sem, ssem):
    c = lax.axis_index("c")
    s = lax.axis_index("s")
    zv = jnp.zeros((16,), jnp.float32)

    def _zrow(i, carry):
        for j in range(DC // 16):
            zbuf[i, pl.ds(j * 16, 16)] = zv
        return carry

    lax.fori_loop(0, ZB, _zrow, 0)

    pltpu.sync_copy(srcp.at[s], src_idx)
    pltpu.sync_copy(dstp.at[s], raw_idx)
    pltpu.sync_copy(slp.at[s], slv)
    pltpu.sync_copy(cnp.at[s], cnv)

    def _chunk(jc, carry):
        ch = c * (NCHUNK // 2) + jc
        for p in range(2):
            zrows = Z1 if p == 0 else Z2
            for off in range(0, zrows, ZB):
                nz = min(ZB, zrows - off)
                pltpu.sync_copy(zbuf.at[pl.ds(0, nz)],
                                acc.at[pl.ds(s * zrows + off, nz)])
            plsc.subcore_barrier()

            cnt = cnv[p, pl.ds(0, 16)][0]
            b0 = slv[p, pl.ds(0, 16)][0]
            pltpu.async_copy(t_hbm.at[ch].at[src_idx.at[b0]], rows.at[0],
                             sem.at[0])

            def _batch(i, b):
                par = i % 2
                npar = 1 - par

                @pl.when(i > 0)
                def _():
                    pltpu.make_async_copy(rows.at[npar],
                                          acc.at[pl.ds(0, BS)],
                                          ssem.at[npar]).wait()

                nb = slv[p, pl.ds(i + 1, 16)][0]
                pltpu.async_copy(t_hbm.at[ch].at[src_idx.at[nb]],
                                 rows.at[npar], sem.at[npar])
                for jj in range(BS // 16):
                    v = raw_idx[b, pl.ds(jj * 16, 16)]
                    tr = (lax.iota(jnp.int32, 16) + 4 * jj) % 64
                    if p == 0:
                        dbuf[par, pl.ds(jj * 16, 16)] = jnp.where(
                            v < THR, v, THR + tr)
                    else:
                        dbuf[par, pl.ds(jj * 16, 16)] = jnp.where(
                            v >= THR, v - THR, P2R + tr)
                pltpu.make_async_copy(t_hbm.at[ch].at[src_idx.at[b]],
                                      rows.at[par], sem.at[par]).wait()
                pltpu.async_copy(rows.at[par], acc.at[dbuf.at[par]],
                                 ssem.at[par], add=True)
                return nb

            bl = lax.fori_loop(0, cnt, _batch, b0)
            parl = cnt % 2

            @pl.when(cnt > 0)
            def _():
                pltpu.make_async_copy(rows.at[1 - parl],
                                      acc.at[pl.ds(0, BS)],
                                      ssem.at[1 - parl]).wait()

            pltpu.make_async_copy(t_hbm.at[ch].at[src_idx.at[bl]],
                                  rows.at[parl], sem.at[parl]).wait()
            plsc.subcore_barrier()
            if p == 0:
                pltpu.sync_copy(acc.at[pl.ds(s * W1R, W1R)],
                                out_hbm.at[ch].at[pl.ds(s * W1R, W1R)])
            else:
                pltpu.sync_copy(acc.at[pl.ds(s * W2R, W2R)],
                                out_hbm.at[ch].at[pl.ds(THR + s * W2R, W2R)])
            plsc.subcore_barrier()
        return carry

    lax.fori_loop(0, NCHUNK // 2, _chunk, 0)


def _propagate(table, srcp, dstp, slp, cnp):
    return pl.kernel(
        _prop_body,
        out_type=jax.ShapeDtypeStruct((NCHUNK, NT, DC), jnp.float32),
        mesh=_sc_mesh,
        scratch_types=[
            pltpu.VMEM((BPT, BS), jnp.int32),
            pltpu.VMEM((BPT, BS), jnp.int32),
            pltpu.VMEM((2, BS), jnp.int32),
            pltpu.VMEM((2, BS, DC), jnp.float32),
            pltpu.VMEM((ZB, DC), jnp.float32),
            pltpu.VMEM((2, BS), jnp.int32),
            pltpu.VMEM((2, 16), jnp.int32),
            pltpu.VMEM_SHARED((A1, DC), jnp.float32),
            pltpu.SemaphoreType.DMA((2,)),
            pltpu.SemaphoreType.DMA((2,)),
        ],
    )(table, srcp, dstp, slp, cnp)


def _deg_body(dstp, deg_out, dst_idx, ones_v, zbuf, acc):
    c = lax.axis_index("c")
    s = lax.axis_index("s")
    zv = jnp.zeros((16,), jnp.float32)
    ov = jnp.ones((16,), jnp.float32)
    for i in range(BS // 16):
        ones_v[pl.ds(i * 16, 16)] = ov

    def _z(i, carry):
        zbuf[pl.ds(i * 16, 16)] = zv
        return carry

    lax.fori_loop(0, RPTD // 16, _z, 0)

    pltpu.sync_copy(dstp.at[s], dst_idx)
    pltpu.sync_copy(zbuf, acc.at[pl.ds(s * RPTD, RPTD)])
    plsc.subcore_barrier()

    def _b(b, carry):
        pltpu.sync_copy(ones_v, acc.at[dst_idx.at[c * (BPT // 2) + b]],
                        add=True)
        return carry

    lax.fori_loop(0, BPT // 2, _b, 0)
    plsc.subcore_barrier()
    pltpu.sync_copy(acc.at[pl.ds(s * RPTD, RPTD)],
                    deg_out.at[c].at[pl.ds(s * RPTD, RPTD)])


def _degrees(dstp):
    return pl.kernel(
        _deg_body,
        out_type=jax.ShapeDtypeStruct((2, NTD), jnp.float32),
        mesh=_sc_mesh,
        scratch_types=[
            pltpu.VMEM((BPT, BS), jnp.int32),
            pltpu.VMEM((BS,), jnp.float32),
            pltpu.VMEM((RPTD,), jnp.float32),
            pltpu.VMEM_SHARED((NTD,), jnp.float32),
        ],
    )(dstp)


def _dis_body(degf_ref, dis_ref):
    d = degf_ref[pl.ds(0, NTD)] + degf_ref[pl.ds(NTD, NTD)]
    dis_ref[...] = jnp.where(d > 0.0, lax.rsqrt(jnp.maximum(d, 1e-12)), 0.0)


_dis_call = pl.pallas_call(
    _dis_body,
    out_shape=jax.ShapeDtypeStruct((NTD,), jnp.float32),
)


def _emb_body(x_ref, w_ref, b_ref, h_ref):
    h_ref[...] = (jnp.dot(x_ref[...], w_ref[...],
                          preferred_element_type=jnp.float32) + b_ref[...])


_emb_call = pl.pallas_call(
    _emb_body,
    grid=(GRID,),
    in_specs=[
        pl.BlockSpec((R, D_IN), lambda i: (i, 0)),
        pl.BlockSpec((D_IN, D_H), lambda i: (0, 0)),
        pl.BlockSpec((D_H,), lambda i: (0,)),
    ],
    out_specs=pl.BlockSpec((R, D_H), lambda i: (i, 0)),
    out_shape=jax.ShapeDtypeStruct((N, D_H), jnp.float32),
)


def _scale_body(h_ref, dis_ref, u_ref):
    hs = h_ref[...] * dis_ref[...]
    for cc in range(NCHUNK):
        u_ref[cc] = hs[:, cc * DC:(cc + 1) * DC]


_scale_call = pl.pallas_call(
    _scale_body,
    grid=(GRID,),
    in_specs=[
        pl.BlockSpec((R, D_H), lambda i: (i, 0)),
        pl.BlockSpec((R, 1), lambda i: (i, 0)),
    ],
    out_specs=pl.BlockSpec((NCHUNK, R, DC), lambda i: (0, i, 0)),
    out_shape=jax.ShapeDtypeStruct((NCHUNK, NT, DC), jnp.float32),
)


def _mm0_body(h_ref, w_ref, s_ref):
    s_ref[...] = jnp.dot(h_ref[...], w_ref[...],
                         preferred_element_type=jnp.float32)


_mm0_call = pl.pallas_call(
    _mm0_body,
    grid=(GRID,),
    in_specs=[
        pl.BlockSpec((R, D_H), lambda i: (i, 0)),
        pl.BlockSpec((D_H, D_H), lambda i: (0, 0)),
    ],
    out_specs=pl.BlockSpec((R, D_H), lambda i: (i, 0)),
    out_shape=jax.ShapeDtypeStruct((N, D_H), jnp.float32),
)


def _mid_body(p_ref, dis_ref, sin_ref, w_ref, sout_ref, v_ref):
    dis = dis_ref[...]
    p = jnp.concatenate([p_ref[cc] for cc in range(NCHUNK)], axis=1)
    t = p * dis
    v = t * dis
    for cc in range(NCHUNK):
        v_ref[cc] = v[:, cc * DC:(cc + 1) * DC]
    sout_ref[...] = sin_ref[...] + jnp.dot(t, w_ref[...],
                                           preferred_element_type=jnp.float32)


_mid_call = pl.pallas_call(
    _mid_body,
    grid=(GRID,),
    in_specs=[
        pl.BlockSpec((NCHUNK, R, DC), lambda i: (0, i, 0)),
        pl.BlockSpec((R, 1), lambda i: (i, 0)),
        pl.BlockSpec((R, D_H), lambda i: (i, 0)),
        pl.BlockSpec((D_H, D_H), lambda i: (0, 0)),
    ],
    out_specs=[
        pl.BlockSpec((R, D_H), lambda i: (i, 0)),
        pl.BlockSpec((NCHUNK, R, DC), lambda i: (0, i, 0)),
    ],
    out_shape=[
        jax.ShapeDtypeStruct((N, D_H), jnp.float32),
        jax.ShapeDtypeStruct((NCHUNK, NT, DC), jnp.float32),
    ],
)


def _post_body(p_ref, dis_ref, sin_ref, w_ref, b_ref, h_ref, w0_ref,
               hout_ref, u_ref, s0_ref):
    dis = dis_ref[...]
    p = jnp.concatenate([p_ref[cc] for cc in range(NCHUNK)], axis=1)
    t = p * dis
    conv = (sin_ref[...]
            + jnp.dot(t, w_ref[...], preferred_element_type=jnp.float32)
            + b_ref[...])
    hn = h_ref[...] + EPS * jnp.tanh(conv)
    hout_ref[...] = hn
    hs = hn * dis
    for cc in range(NCHUNK):
        u_ref[cc] = hs[:, cc * DC:(cc + 1) * DC]
    s0_ref[...] = jnp.dot(hn, w0_ref[...], preferred_element_type=jnp.float32)


_post_call = pl.pallas_call(
    _post_body,
    grid=(GRID,),
    in_specs=[
        pl.BlockSpec((NCHUNK, R, DC), lambda i: (0, i, 0)),
        pl.BlockSpec((R, 1), lambda i: (i, 0)),
        pl.BlockSpec((R, D_H), lambda i: (i, 0)),
        pl.BlockSpec((D_H, D_H), lambda i: (0, 0)),
        pl.BlockSpec((D_H,), lambda i: (0,)),
        pl.BlockSpec((R, D_H), lambda i: (i, 0)),
        pl.BlockSpec((D_H, D_H), lambda i: (0, 0)),
    ],
    out_specs=[
        pl.BlockSpec((R, D_H), lambda i: (i, 0)),
        pl.BlockSpec((NCHUNK, R, DC), lambda i: (0, i, 0)),
        pl.BlockSpec((R, D_H), lambda i: (i, 0)),
    ],
    out_shape=[
        jax.ShapeDtypeStruct((N, D_H), jnp.float32),
        jax.ShapeDtypeStruct((NCHUNK, NT, DC), jnp.float32),
        jax.ShapeDtypeStruct((N, D_H), jnp.float32),
    ],
)


def _ro_body(h_ref, w_ref, b_ref, y_ref):
    y_ref[...] = (jnp.dot(h_ref[...], w_ref[...],
                          preferred_element_type=jnp.float32) + b_ref[...])


_ro_call = pl.pallas_call(
    _ro_body,
    grid=(GRID,),
    in_specs=[
        pl.BlockSpec((R, D_H), lambda i: (i, 0)),
        pl.BlockSpec((D_H, D_OUT), lambda i: (0, 0)),
        pl.BlockSpec((D_OUT,), lambda i: (0,)),
    ],
    out_specs=pl.BlockSpec((R, D_OUT), lambda i: (i, 0)),
    out_shape=jax.ShapeDtypeStruct((N, D_OUT), jnp.float32),
)


def kernel(x, edge_index, delta_t, emb_w, emb_b, lins_w, conv_b, ro_w, ro_b):
    src = edge_index[0]
    dst = edge_index[1]
    bit = (dst >= THR).astype(jnp.int32)
    g = jnp.argsort(bit, stable=True)
    fill = (N + (jnp.arange(EP - E, dtype=jnp.int32) % 64)).astype(jnp.int32)
    src_part = jnp.concatenate([src[g], fill])
    dst_part = jnp.concatenate([dst[g], fill])
    bmin = dst_part.reshape(NB, BS).min(axis=1)
    bmax = dst_part.reshape(NB, BS).max(axis=1)
    real = jnp.arange(NB) < NBR
    f1 = ((bmin < THR) & real).astype(jnp.int32)
    f2 = ((bmax >= THR) & real).astype(jnp.int32)
    af = jnp.stack([f1, f2], 1).reshape(BPT, NTILE, 2).transpose(1, 2, 0)
    slp = jnp.pad(jnp.argsort(1 - af, axis=-1),
                  ((0, 0), (0, 0), (0, BS - BPT))).astype(jnp.int32)
    cnp = jnp.pad(af.sum(-1, keepdims=True),
                  ((0, 0), (0, 0), (0, 15))).astype(jnp.int32)

    def _tiled(a):
        return a.reshape(BPT, NTILE, BS).transpose(1, 0, 2)

    srcp = _tiled(src_part)
    dstp = _tiled(dst_part)

    degp = _degrees(dstp)
    dis = _dis_call(degp.reshape(2 * NTD))
    dis2 = dis.reshape(NTD, 1)

    h0 = _emb_call(x, emb_w, emb_b)

    w0 = lins_w[0]
    w1 = lins_w[1]
    w2 = lins_w[2]

    u0 = _scale_call(h0, dis2)
    s00 = _mm0_call(h0, w0)

    def _step(_, carry):
        h, u, s0 = carry
        p1 = _propagate(u, srcp, dstp, slp, cnp)
        s1, v = _mid_call(p1, dis2, s0, w1)
        p2 = _propagate(v, srcp, dstp, slp, cnp)
        return _post_call(p2, dis2, s1, w2, conv_b, h, w0)

    h, _, _ = lax.fori_loop(0, delta_t, _step, (h0, u0, s00))
    y = _ro_call(h, ro_w, ro_b)
    return (y, h)
